# baseline (speedup 1.0000x reference)
import functools

import jax
import jax.numpy as jnp
from jax import lax
from jax.experimental import pallas as pl
from jax.experimental.pallas import tpu as pltpu
from jax.experimental.pallas import tpu_sc as plsc

D = 768
E = 64
DFF = 768
CAP_FACTOR = 1.25
THRESH = 0.2
LOSS_COEF = 0.01
TB = 256
TK = 64


def _routing_a(x_ref, wg_ref, u_ref,
               idx1_ref, p1_ref, keep1_ref, g1_ref,
               idx2_ref, g2_ref, keep2r_ref, cntf_ref, gsum_ref,
               C: int):
    j = pl.program_id(0)

    @pl.when(j == 0)
    def _():
        cntf_ref[...] = jnp.zeros_like(cntf_ref)
        gsum_ref[...] = jnp.zeros_like(gsum_ref)

    x = x_ref[...]
    logits = jax.lax.dot_general(x, wg_ref[...], (((1,), (0,)), ((), ())),
                                 preferred_element_type=jnp.float32)
    m = jnp.max(logits, axis=-1, keepdims=True)
    ex = jnp.exp(logits - m)
    raw = ex / jnp.sum(ex, axis=-1, keepdims=True)

    lane = jax.lax.broadcasted_iota(jnp.int32, (TB, E), 1)
    g1v = jnp.max(raw, axis=-1, keepdims=True)
    i1 = jnp.min(jnp.where(raw == g1v, lane, E), axis=-1, keepdims=True)
    mask1 = (lane == i1).astype(jnp.float32)
    wo1 = raw * (1.0 - mask1)
    g2v = jnp.max(wo1, axis=-1, keepdims=True)
    i2 = jnp.min(jnp.where(wo1 == g2v, lane, E), axis=-1, keepdims=True)

    denom = g1v + g2v + 1e-9
    g1 = g1v / denom
    g2 = g2v / denom
    keep2r = (u_ref[...] < jnp.clip(g2 / THRESH, 0.0, 1.0)).astype(jnp.float32)

    row = jax.lax.broadcasted_iota(jnp.int32, (TB, TB), 0)
    col = jax.lax.broadcasted_iota(jnp.int32, (TB, TB), 1)
    lstrict = (col < row).astype(jnp.float32)
    pos1 = jax.lax.dot_general(lstrict, mask1, (((1,), (0,)), ((), ())),
                               preferred_element_type=jnp.float32) + cntf_ref[...]
    cntf_ref[...] += jnp.sum(mask1, axis=0, keepdims=True)
    gsum_ref[...] += jnp.sum(raw, axis=0, keepdims=True)
    m1c = mask1 * (pos1 < C).astype(jnp.float32)
    keep1 = jnp.sum(m1c, axis=-1, keepdims=True)
    p1 = jnp.sum(pos1 * m1c, axis=-1, keepdims=True).astype(jnp.int32)

    idx1_ref[...] = i1
    idx2_ref[...] = i2
    p1_ref[...] = p1
    keep1_ref[...] = keep1
    g1_ref[...] = g1
    g2_ref[...] = g2
    keep2r_ref[...] = keep2r


def _routing_b(idx2_ref, keep2r_ref, cntf_ref, p2_ref, keep2_ref, cnt2c_ref,
               cnt2_ref, C: int):
    j = pl.program_id(0)

    @pl.when(j == 0)
    def _():
        cnt2_ref[...] = jnp.zeros_like(cnt2_ref)
        cnt2c_ref[...] = jnp.zeros_like(cnt2c_ref)

    lane = jax.lax.broadcasted_iota(jnp.int32, (TB, E), 1)
    mask2 = (lane == idx2_ref[...]).astype(jnp.float32) * keep2r_ref[...]
    cnt1 = jnp.minimum(cntf_ref[...], float(C))
    row = jax.lax.broadcasted_iota(jnp.int32, (TB, TB), 0)
    col = jax.lax.broadcasted_iota(jnp.int32, (TB, TB), 1)
    lstrict = (col < row).astype(jnp.float32)
    pos2 = (jax.lax.dot_general(lstrict, mask2, (((1,), (0,)), ((), ())),
                                preferred_element_type=jnp.float32)
            + cnt2_ref[...] + cnt1)
    cnt2_ref[...] += jnp.sum(mask2, axis=0, keepdims=True)
    m2c = mask2 * (pos2 < C).astype(jnp.float32)
    cnt2c_ref[...] += jnp.sum(m2c, axis=0, keepdims=True)
    keep2_ref[...] = jnp.sum(m2c, axis=-1, keepdims=True)
    p2_ref[...] = jnp.sum(pos2 * m2c, axis=-1, keepdims=True).astype(jnp.int32)


def _expert_ffn(xd_ref, wslot_ref, w1_ref, b1_ref, w2_ref, b2_ref, odw_ref):
    e = pl.program_id(0)

    @pl.when(e < E)
    def _():
        h = jax.lax.dot_general(xd_ref[...].astype(jnp.bfloat16), w1_ref[0],
                                (((1,), (0,)), ((), ())),
                                preferred_element_type=jnp.float32)
        h = jnp.maximum(h + b1_ref[0], 0.0).astype(jnp.bfloat16)
        o = jax.lax.dot_general(h, w2_ref[0], (((1,), (0,)), ((), ())),
                                preferred_element_type=jnp.float32) + b2_ref[0]
        odw_ref[...] = o * wslot_ref[0]

    @pl.when(e == E)
    def _():
        odw_ref[...] = jnp.zeros_like(odw_ref)


def _make_sc_gather(n_rows, table_rows):
    nw = 32
    per_w = n_rows // nw
    chunk = 64
    nch = per_w // chunk
    mesh = plsc.VectorSubcoreMesh(core_axis_name="c", subcore_axis_name="s")

    @functools.partial(
        pl.kernel, mesh=mesh,
        out_type=jax.ShapeDtypeStruct((n_rows, D), jnp.float32),
        scratch_types=[
            pltpu.VMEM((per_w,), jnp.int32),
            pltpu.VMEM((chunk, D), jnp.float32),
            pltpu.VMEM((chunk, D), jnp.float32),
            pltpu.SemaphoreType.DMA,
            pltpu.SemaphoreType.DMA,
        ],
    )
    def gather(table_hbm, idx_hbm, out_hbm, idx_v, rb0, rb1, sem0, sem1):
        wid = lax.axis_index("s") * 2 + lax.axis_index("c")
        base = wid * per_w
        pltpu.sync_copy(idx_hbm.at[pl.ds(base, per_w)], idx_v)
        bufs = (rb0, rb1)
        sems = (sem0, sem1)

        def fire(j):
            return pltpu.async_copy(
                table_hbm.at[idx_v.at[pl.ds(j * chunk, chunk)]],
                bufs[j % 2], sems[j % 2])

        handles = [fire(0)]
        for j in range(nch):
            handles[j].wait()
            if j + 1 < nch:
                handles.append(fire(j + 1))
            pltpu.sync_copy(bufs[j % 2],
                            out_hbm.at[pl.ds(base + j * chunk, chunk)])

    return gather


def _add_halves(a_ref, b_ref, o_ref):
    o_ref[...] = a_ref[...] + b_ref[...]


def kernel(hidden_states, w_gate, W1, b1, W2, b2):
    orig_shape = hidden_states.shape
    x = hidden_states.reshape(-1, hidden_states.shape[-1])
    T = x.shape[0]
    C = int(2 * CAP_FACTOR * T / E)
    nb = T // TB

    u = jax.random.uniform(jax.random.key(1), (T,), dtype=jnp.float32)
    u2 = u.reshape(T, 1)

    f32 = jnp.float32
    i32 = jnp.int32
    outs_a = (
        jax.ShapeDtypeStruct((T, 1), i32),
        jax.ShapeDtypeStruct((T, 1), i32),
        jax.ShapeDtypeStruct((T, 1), f32),
        jax.ShapeDtypeStruct((T, 1), f32),
        jax.ShapeDtypeStruct((T, 1), i32),
        jax.ShapeDtypeStruct((T, 1), f32),
        jax.ShapeDtypeStruct((T, 1), f32),
        jax.ShapeDtypeStruct((1, E), f32),
        jax.ShapeDtypeStruct((1, E), f32),
    )
    tb_spec = pl.BlockSpec((TB, 1), lambda j: (j, 0))
    acc_spec = pl.BlockSpec((1, E), lambda j: (0, 0))
    idx1, p1, keep1, g1, idx2, g2, keep2r, cntf, gsum = pl.pallas_call(
        lambda *refs: _routing_a(*refs, C=C),
        grid=(nb,),
        in_specs=[
            pl.BlockSpec((TB, D), lambda j: (j, 0)),
            pl.BlockSpec((D, E), lambda j: (0, 0)),
            tb_spec,
        ],
        out_specs=(tb_spec,) * 7 + (acc_spec, acc_spec),
        out_shape=outs_a,
        compiler_params=pltpu.CompilerParams(
            dimension_semantics=("arbitrary",)),
    )(x, w_gate, u2)

    p2, keep2, cnt2c = pl.pallas_call(
        lambda *refs: _routing_b(*refs, C=C),
        grid=(nb,),
        in_specs=[tb_spec, tb_spec, acc_spec],
        out_specs=(tb_spec, tb_spec, acc_spec),
        out_shape=(jax.ShapeDtypeStruct((T, 1), i32),
                   jax.ShapeDtypeStruct((T, 1), f32),
                   jax.ShapeDtypeStruct((1, E), f32)),
        scratch_shapes=[pltpu.VMEM((1, E), f32)],
        compiler_params=pltpu.CompilerParams(
            dimension_semantics=("arbitrary",)),
    )(idx2, keep2r, cntf)

    tid = jnp.arange(T, dtype=i32)
    i1 = idx1[:, 0]
    i2 = idx2[:, 0]
    p1f = p1[:, 0]
    p2f = p2[:, 0]
    e1s = jnp.where(keep1[:, 0] > 0, i1, E)
    e2s = jnp.where(keep2[:, 0] > 0, i2, E)
    assign = jnp.full((E, C), -1, i32)
    assign = assign.at[e1s, p1f].set(tid, mode="drop")
    assign = assign.at[e2s, p2f].set(tid, mode="drop")
    wslot = jnp.zeros((E, C), f32)
    wslot = wslot.at[e1s, p1f].set(g1[:, 0], mode="drop")
    wslot = wslot.at[e2s, p2f].set(g2[:, 0], mode="drop")
    assign_flat = jnp.maximum(assign.reshape(E * C), 0)
    xd = _make_sc_gather(E * C, T)(x, assign_flat)

    zrow = E * C
    odw = pl.pallas_call(
        _expert_ffn,
        grid=(E + 1,),
        in_specs=[
            pl.BlockSpec((C, D), lambda e: (e - e // E, 0)),
            pl.BlockSpec((1, C, 1), lambda e: (e - e // E, 0, 0)),
            pl.BlockSpec((1, D, DFF), lambda e: (e - e // E, 0, 0)),
            pl.BlockSpec((1, 1, DFF), lambda e: (e - e // E, 0, 0)),
            pl.BlockSpec((1, DFF, D), lambda e: (e - e // E, 0, 0)),
            pl.BlockSpec((1, 1, D), lambda e: (e - e // E, 0, 0)),
        ],
        out_specs=pl.BlockSpec((C, D), lambda e: (e, 0)),
        out_shape=jax.ShapeDtypeStruct(((E + 1) * C, D), f32),
        compiler_params=pltpu.CompilerParams(
            dimension_semantics=("arbitrary",)),
    )(xd, wslot.reshape(E, C, 1),
      W1.astype(jnp.bfloat16), b1.reshape(E, 1, DFF),
      W2.astype(jnp.bfloat16), b2.reshape(E, 1, D))

    loc1 = jnp.where(keep1[:, 0] > 0, i1 * C + p1f, zrow)
    loc2 = jnp.where(keep2[:, 0] > 0, i2 * C + p2f, zrow)
    locs = jnp.concatenate([loc1, loc2])
    y12 = _make_sc_gather(2 * T, (E + 1) * C)(odw, locs)
    yb = T // TB
    y = pl.pallas_call(
        _add_halves,
        grid=(yb,),
        in_specs=[
            pl.BlockSpec((TB, D), lambda j: (j, 0)),
            pl.BlockSpec((TB, D), lambda j: (j + yb, 0)),
        ],
        out_specs=pl.BlockSpec((TB, D), lambda j: (j, 0)),
        out_shape=jax.ShapeDtypeStruct((T, D), f32),
    )(y12, y12)

    density = cntf[0] / T
    density_proxy = gsum[0] / T
    loss = jnp.mean(density * density_proxy) * (E * E) * LOSS_COEF
    return y.reshape(orig_shape), loss


# speedup vs baseline: 1.1812x; 1.1812x over previous
import functools

import jax
import jax.numpy as jnp
from jax import lax
from jax.experimental import pallas as pl
from jax.experimental.pallas import tpu as pltpu
from jax.experimental.pallas import tpu_sc as plsc

D = 768
E = 64
DFF = 768
CAP_FACTOR = 1.25
THRESH = 0.2
LOSS_COEF = 0.01
TB = 256
TK = 64


def _routing_a(x_ref, wg_ref, u_ref,
               idx1_ref, p1_ref, keep1_ref, g1_ref,
               idx2_ref, g2_ref, keep2r_ref, cntf_ref, gsum_ref,
               C: int):
    j = pl.program_id(0)

    @pl.when(j == 0)
    def _():
        cntf_ref[...] = jnp.zeros_like(cntf_ref)
        gsum_ref[...] = jnp.zeros_like(gsum_ref)

    x = x_ref[...]
    logits = jax.lax.dot_general(x, wg_ref[...], (((1,), (0,)), ((), ())),
                                 preferred_element_type=jnp.float32)
    m = jnp.max(logits, axis=-1, keepdims=True)
    ex = jnp.exp(logits - m)
    raw = ex / jnp.sum(ex, axis=-1, keepdims=True)

    lane = jax.lax.broadcasted_iota(jnp.int32, (TB, E), 1)
    g1v = jnp.max(raw, axis=-1, keepdims=True)
    i1 = jnp.min(jnp.where(raw == g1v, lane, E), axis=-1, keepdims=True)
    mask1 = (lane == i1).astype(jnp.float32)
    wo1 = raw * (1.0 - mask1)
    g2v = jnp.max(wo1, axis=-1, keepdims=True)
    i2 = jnp.min(jnp.where(wo1 == g2v, lane, E), axis=-1, keepdims=True)

    denom = g1v + g2v + 1e-9
    g1 = g1v / denom
    g2 = g2v / denom
    keep2r = (u_ref[...] < jnp.clip(g2 / THRESH, 0.0, 1.0)).astype(jnp.float32)

    row = jax.lax.broadcasted_iota(jnp.int32, (TB, TB), 0)
    col = jax.lax.broadcasted_iota(jnp.int32, (TB, TB), 1)
    lstrict = (col < row).astype(jnp.float32)
    pos1 = jax.lax.dot_general(lstrict, mask1, (((1,), (0,)), ((), ())),
                               preferred_element_type=jnp.float32) + cntf_ref[...]
    cntf_ref[...] += jnp.sum(mask1, axis=0, keepdims=True)
    gsum_ref[...] += jnp.sum(raw, axis=0, keepdims=True)
    m1c = mask1 * (pos1 < C).astype(jnp.float32)
    keep1 = jnp.sum(m1c, axis=-1, keepdims=True)
    p1 = jnp.sum(pos1 * m1c, axis=-1, keepdims=True).astype(jnp.int32)

    idx1_ref[...] = i1
    idx2_ref[...] = i2
    p1_ref[...] = p1
    keep1_ref[...] = keep1
    g1_ref[...] = g1
    g2_ref[...] = g2
    keep2r_ref[...] = keep2r


def _routing_b(idx2_ref, keep2r_ref, cntf_ref, p2_ref, keep2_ref, cnt2c_ref,
               cnt2_ref, C: int):
    j = pl.program_id(0)

    @pl.when(j == 0)
    def _():
        cnt2_ref[...] = jnp.zeros_like(cnt2_ref)
        cnt2c_ref[...] = jnp.zeros_like(cnt2c_ref)

    lane = jax.lax.broadcasted_iota(jnp.int32, (TB, E), 1)
    mask2 = (lane == idx2_ref[...]).astype(jnp.float32) * keep2r_ref[...]
    cnt1 = jnp.minimum(cntf_ref[...], float(C))
    row = jax.lax.broadcasted_iota(jnp.int32, (TB, TB), 0)
    col = jax.lax.broadcasted_iota(jnp.int32, (TB, TB), 1)
    lstrict = (col < row).astype(jnp.float32)
    pos2 = (jax.lax.dot_general(lstrict, mask2, (((1,), (0,)), ((), ())),
                                preferred_element_type=jnp.float32)
            + cnt2_ref[...] + cnt1)
    cnt2_ref[...] += jnp.sum(mask2, axis=0, keepdims=True)
    m2c = mask2 * (pos2 < C).astype(jnp.float32)
    cnt2c_ref[...] += jnp.sum(m2c, axis=0, keepdims=True)
    keep2_ref[...] = jnp.sum(m2c, axis=-1, keepdims=True)
    p2_ref[...] = jnp.sum(pos2 * m2c, axis=-1, keepdims=True).astype(jnp.int32)


def _expert_ffn(xd_ref, wslot_ref, w1_ref, b1_ref, w2_ref, b2_ref, odw_ref):
    e = pl.program_id(0)

    @pl.when(e < E)
    def _():
        h = jax.lax.dot_general(xd_ref[...].astype(jnp.bfloat16),
                                w1_ref[0].astype(jnp.bfloat16),
                                (((1,), (0,)), ((), ())),
                                preferred_element_type=jnp.float32)
        h = jnp.maximum(h + b1_ref[0], 0.0).astype(jnp.bfloat16)
        o = jax.lax.dot_general(h, w2_ref[0].astype(jnp.bfloat16),
                                (((1,), (0,)), ((), ())),
                                preferred_element_type=jnp.float32) + b2_ref[0]
        odw_ref[...] = o * wslot_ref[0]

    @pl.when(e == E)
    def _():
        odw_ref[...] = jnp.zeros_like(odw_ref)


def _make_sc_gather(n_rows, table_rows):
    nw = 32
    per_w = n_rows // nw
    chunk = 64
    nch = per_w // chunk
    mesh = plsc.VectorSubcoreMesh(core_axis_name="c", subcore_axis_name="s")

    @functools.partial(
        pl.kernel, mesh=mesh,
        out_type=jax.ShapeDtypeStruct((n_rows, D), jnp.float32),
        scratch_types=[
            pltpu.VMEM((per_w,), jnp.int32),
            pltpu.VMEM((chunk, D), jnp.float32),
            pltpu.VMEM((chunk, D), jnp.float32),
            pltpu.SemaphoreType.DMA,
            pltpu.SemaphoreType.DMA,
        ],
    )
    def gather(table_hbm, idx_hbm, out_hbm, idx_v, rb0, rb1, sem0, sem1):
        wid = lax.axis_index("s") * 2 + lax.axis_index("c")
        base = wid * per_w
        pltpu.sync_copy(idx_hbm.at[pl.ds(base, per_w)], idx_v)
        bufs = (rb0, rb1)
        sems = (sem0, sem1)

        def fire(j):
            return pltpu.async_copy(
                table_hbm.at[idx_v.at[pl.ds(j * chunk, chunk)]],
                bufs[j % 2], sems[j % 2])

        handles = [fire(0)]
        for j in range(nch):
            handles[j].wait()
            if j + 1 < nch:
                handles.append(fire(j + 1))
            pltpu.sync_copy(bufs[j % 2],
                            out_hbm.at[pl.ds(base + j * chunk, chunk)])

    return gather


def _add_halves(a_ref, b_ref, o_ref):
    o_ref[...] = a_ref[...] + b_ref[...]


def kernel(hidden_states, w_gate, W1, b1, W2, b2):
    orig_shape = hidden_states.shape
    x = hidden_states.reshape(-1, hidden_states.shape[-1])
    T = x.shape[0]
    C = int(2 * CAP_FACTOR * T / E)
    nb = T // TB

    u = jax.random.uniform(jax.random.key(1), (T,), dtype=jnp.float32)
    u2 = u.reshape(T, 1)

    f32 = jnp.float32
    i32 = jnp.int32
    outs_a = (
        jax.ShapeDtypeStruct((T, 1), i32),
        jax.ShapeDtypeStruct((T, 1), i32),
        jax.ShapeDtypeStruct((T, 1), f32),
        jax.ShapeDtypeStruct((T, 1), f32),
        jax.ShapeDtypeStruct((T, 1), i32),
        jax.ShapeDtypeStruct((T, 1), f32),
        jax.ShapeDtypeStruct((T, 1), f32),
        jax.ShapeDtypeStruct((1, E), f32),
        jax.ShapeDtypeStruct((1, E), f32),
    )
    tb_spec = pl.BlockSpec((TB, 1), lambda j: (j, 0))
    acc_spec = pl.BlockSpec((1, E), lambda j: (0, 0))
    idx1, p1, keep1, g1, idx2, g2, keep2r, cntf, gsum = pl.pallas_call(
        lambda *refs: _routing_a(*refs, C=C),
        grid=(nb,),
        in_specs=[
            pl.BlockSpec((TB, D), lambda j: (j, 0)),
            pl.BlockSpec((D, E), lambda j: (0, 0)),
            tb_spec,
        ],
        out_specs=(tb_spec,) * 7 + (acc_spec, acc_spec),
        out_shape=outs_a,
        compiler_params=pltpu.CompilerParams(
            dimension_semantics=("arbitrary",)),
    )(x, w_gate, u2)

    p2, keep2, cnt2c = pl.pallas_call(
        lambda *refs: _routing_b(*refs, C=C),
        grid=(nb,),
        in_specs=[tb_spec, tb_spec, acc_spec],
        out_specs=(tb_spec, tb_spec, acc_spec),
        out_shape=(jax.ShapeDtypeStruct((T, 1), i32),
                   jax.ShapeDtypeStruct((T, 1), f32),
                   jax.ShapeDtypeStruct((1, E), f32)),
        scratch_shapes=[pltpu.VMEM((1, E), f32)],
        compiler_params=pltpu.CompilerParams(
            dimension_semantics=("arbitrary",)),
    )(idx2, keep2r, cntf)

    tid = jnp.arange(T, dtype=i32)
    i1 = idx1[:, 0]
    i2 = idx2[:, 0]
    p1f = p1[:, 0]
    p2f = p2[:, 0]
    e1s = jnp.where(keep1[:, 0] > 0, i1, E)
    e2s = jnp.where(keep2[:, 0] > 0, i2, E)
    assign = jnp.full((E, C), -1, i32)
    assign = assign.at[e1s, p1f].set(tid, mode="drop")
    assign = assign.at[e2s, p2f].set(tid, mode="drop")
    wslot = jnp.zeros((E, C), f32)
    wslot = wslot.at[e1s, p1f].set(g1[:, 0], mode="drop")
    wslot = wslot.at[e2s, p2f].set(g2[:, 0], mode="drop")
    assign_flat = jnp.maximum(assign.reshape(E * C), 0)
    xd = _make_sc_gather(E * C, T)(x, assign_flat)

    zrow = E * C
    odw = pl.pallas_call(
        _expert_ffn,
        grid=(E + 1,),
        in_specs=[
            pl.BlockSpec((C, D), lambda e: (e - e // E, 0)),
            pl.BlockSpec((1, C, 1), lambda e: (e - e // E, 0, 0)),
            pl.BlockSpec((1, D, DFF), lambda e: (e - e // E, 0, 0)),
            pl.BlockSpec((1, 1, DFF), lambda e: (e - e // E, 0, 0)),
            pl.BlockSpec((1, DFF, D), lambda e: (e - e // E, 0, 0)),
            pl.BlockSpec((1, 1, D), lambda e: (e - e // E, 0, 0)),
        ],
        out_specs=pl.BlockSpec((C, D), lambda e: (e, 0)),
        out_shape=jax.ShapeDtypeStruct(((E + 1) * C, D), f32),
        compiler_params=pltpu.CompilerParams(
            dimension_semantics=("arbitrary",)),
    )(xd, wslot.reshape(E, C, 1),
      W1, b1.reshape(E, 1, DFF),
      W2, b2.reshape(E, 1, D))

    loc1 = jnp.where(keep1[:, 0] > 0, i1 * C + p1f, zrow)
    loc2 = jnp.where(keep2[:, 0] > 0, i2 * C + p2f, zrow)
    locs = jnp.concatenate([loc1, loc2])
    y12 = _make_sc_gather(2 * T, (E + 1) * C)(odw, locs)
    yb = T // TB
    y = pl.pallas_call(
        _add_halves,
        grid=(yb,),
        in_specs=[
            pl.BlockSpec((TB, D), lambda j: (j, 0)),
            pl.BlockSpec((TB, D), lambda j: (j + yb, 0)),
        ],
        out_specs=pl.BlockSpec((TB, D), lambda j: (j, 0)),
        out_shape=jax.ShapeDtypeStruct((T, D), f32),
    )(y12, y12)

    density = cntf[0] / T
    density_proxy = gsum[0] / T
    loss = jnp.mean(density * density_proxy) * (E * E) * LOSS_COEF
    return y.reshape(orig_shape), loss


# speedup vs baseline: 1.5980x; 1.3528x over previous
import functools

import jax
import jax.numpy as jnp
from jax import lax
from jax.experimental import pallas as pl
from jax.experimental.pallas import tpu as pltpu
from jax.experimental.pallas import tpu_sc as plsc

D = 768
E = 64
DFF = 768
CAP_FACTOR = 1.25
THRESH = 0.2
LOSS_COEF = 0.01
TB = 256
TK = 64


def _routing_a(x_ref, wg_ref, u_ref,
               idx1_ref, p1_ref, keep1_ref, g1_ref,
               idx2_ref, g2_ref, keep2r_ref, cntf_ref, gsum_ref,
               C: int):
    j = pl.program_id(0)

    @pl.when(j == 0)
    def _():
        cntf_ref[...] = jnp.zeros_like(cntf_ref)
        gsum_ref[...] = jnp.zeros_like(gsum_ref)

    x = x_ref[...]
    logits = jax.lax.dot_general(x, wg_ref[...], (((1,), (0,)), ((), ())),
                                 preferred_element_type=jnp.float32)
    m = jnp.max(logits, axis=-1, keepdims=True)
    ex = jnp.exp(logits - m)
    raw = ex / jnp.sum(ex, axis=-1, keepdims=True)

    lane = jax.lax.broadcasted_iota(jnp.int32, (TB, E), 1)
    g1v = jnp.max(raw, axis=-1, keepdims=True)
    i1 = jnp.min(jnp.where(raw == g1v, lane, E), axis=-1, keepdims=True)
    mask1 = (lane == i1).astype(jnp.float32)
    wo1 = raw * (1.0 - mask1)
    g2v = jnp.max(wo1, axis=-1, keepdims=True)
    i2 = jnp.min(jnp.where(wo1 == g2v, lane, E), axis=-1, keepdims=True)

    denom = g1v + g2v + 1e-9
    g1 = g1v / denom
    g2 = g2v / denom
    keep2r = (u_ref[...] < jnp.clip(g2 / THRESH, 0.0, 1.0)).astype(jnp.float32)

    row = jax.lax.broadcasted_iota(jnp.int32, (TB, TB), 0)
    col = jax.lax.broadcasted_iota(jnp.int32, (TB, TB), 1)
    lstrict = (col < row).astype(jnp.float32)
    pos1 = jax.lax.dot_general(lstrict, mask1, (((1,), (0,)), ((), ())),
                               preferred_element_type=jnp.float32) + cntf_ref[...]
    cntf_ref[...] += jnp.sum(mask1, axis=0, keepdims=True)
    gsum_ref[...] += jnp.sum(raw, axis=0, keepdims=True)
    m1c = mask1 * (pos1 < C).astype(jnp.float32)
    keep1 = jnp.sum(m1c, axis=-1, keepdims=True)
    p1 = jnp.sum(pos1 * m1c, axis=-1, keepdims=True).astype(jnp.int32)

    idx1_ref[...] = i1
    idx2_ref[...] = i2
    p1_ref[...] = p1
    keep1_ref[...] = keep1
    g1_ref[...] = g1
    g2_ref[...] = g2
    keep2r_ref[...] = keep2r


def _routing_b(idx2_ref, keep2r_ref, cntf_ref, p2_ref, keep2_ref, cnt2c_ref,
               cnt2_ref, C: int):
    j = pl.program_id(0)

    @pl.when(j == 0)
    def _():
        cnt2_ref[...] = jnp.zeros_like(cnt2_ref)
        cnt2c_ref[...] = jnp.zeros_like(cnt2c_ref)

    lane = jax.lax.broadcasted_iota(jnp.int32, (TB, E), 1)
    mask2 = (lane == idx2_ref[...]).astype(jnp.float32) * keep2r_ref[...]
    cnt1 = jnp.minimum(cntf_ref[...], float(C))
    row = jax.lax.broadcasted_iota(jnp.int32, (TB, TB), 0)
    col = jax.lax.broadcasted_iota(jnp.int32, (TB, TB), 1)
    lstrict = (col < row).astype(jnp.float32)
    pos2 = (jax.lax.dot_general(lstrict, mask2, (((1,), (0,)), ((), ())),
                                preferred_element_type=jnp.float32)
            + cnt2_ref[...] + cnt1)
    cnt2_ref[...] += jnp.sum(mask2, axis=0, keepdims=True)
    m2c = mask2 * (pos2 < C).astype(jnp.float32)
    cnt2c_ref[...] += jnp.sum(m2c, axis=0, keepdims=True)
    keep2_ref[...] = jnp.sum(m2c, axis=-1, keepdims=True)
    p2_ref[...] = jnp.sum(pos2 * m2c, axis=-1, keepdims=True).astype(jnp.int32)


def _expert_ffn(xd_ref, wslot_ref, w1_ref, b1_ref, w2_ref, b2_ref, odw_ref):
    e = pl.program_id(0)

    @pl.when(e < E)
    def _():
        h = jax.lax.dot_general(xd_ref[...].astype(jnp.bfloat16),
                                w1_ref[0].astype(jnp.bfloat16),
                                (((1,), (0,)), ((), ())),
                                preferred_element_type=jnp.float32)
        h = jnp.maximum(h + b1_ref[0], 0.0).astype(jnp.bfloat16)
        o = jax.lax.dot_general(h, w2_ref[0].astype(jnp.bfloat16),
                                (((1,), (0,)), ((), ())),
                                preferred_element_type=jnp.float32) + b2_ref[0]
        odw_ref[...] = o * wslot_ref[0]

    @pl.when(e == E)
    def _():
        odw_ref[...] = jnp.zeros_like(odw_ref)


def _make_sc_gather(n_rows, table_rows):
    nw = 32
    per_w = n_rows // nw
    chunk = 64
    nch = per_w // chunk
    mesh = plsc.VectorSubcoreMesh(core_axis_name="c", subcore_axis_name="s")

    @functools.partial(
        pl.kernel, mesh=mesh,
        out_type=jax.ShapeDtypeStruct((n_rows, D), jnp.float32),
        scratch_types=[
            pltpu.VMEM((per_w,), jnp.int32),
            pltpu.VMEM((chunk, D), jnp.float32),
            pltpu.VMEM((chunk, D), jnp.float32),
            pltpu.SemaphoreType.DMA,
            pltpu.SemaphoreType.DMA,
        ],
    )
    def gather(table_hbm, idx_hbm, out_hbm, idx_v, rb0, rb1, sem0, sem1):
        wid = lax.axis_index("s") * 2 + lax.axis_index("c")
        base = wid * per_w
        pltpu.sync_copy(idx_hbm.at[pl.ds(base, per_w)], idx_v)
        bufs = (rb0, rb1)
        sems = (sem0, sem1)

        def fire(j):
            return pltpu.async_copy(
                table_hbm.at[idx_v.at[pl.ds(j * chunk, chunk)]],
                bufs[j % 2], sems[j % 2])

        handles = [fire(0)]
        for j in range(nch):
            handles[j].wait()
            if j + 1 < nch:
                handles.append(fire(j + 1))
            pltpu.sync_copy(bufs[j % 2],
                            out_hbm.at[pl.ds(base + j * chunk, chunk)])

    return gather


def _add_halves(a_ref, b_ref, o_ref):
    o_ref[...] = a_ref[...] + b_ref[...]


def kernel(hidden_states, w_gate, W1, b1, W2, b2):
    orig_shape = hidden_states.shape
    x = hidden_states.reshape(-1, hidden_states.shape[-1])
    T = x.shape[0]
    C = int(2 * CAP_FACTOR * T / E)
    nb = T // TB

    u = jax.random.uniform(jax.random.key(1), (T,), dtype=jnp.float32)
    u2 = u.reshape(T, 1)

    f32 = jnp.float32
    i32 = jnp.int32
    outs_a = (
        jax.ShapeDtypeStruct((T, 1), i32),
        jax.ShapeDtypeStruct((T, 1), i32),
        jax.ShapeDtypeStruct((T, 1), f32),
        jax.ShapeDtypeStruct((T, 1), f32),
        jax.ShapeDtypeStruct((T, 1), i32),
        jax.ShapeDtypeStruct((T, 1), f32),
        jax.ShapeDtypeStruct((T, 1), f32),
        jax.ShapeDtypeStruct((1, E), f32),
        jax.ShapeDtypeStruct((1, E), f32),
    )
    tb_spec = pl.BlockSpec((TB, 1), lambda j: (j, 0))
    acc_spec = pl.BlockSpec((1, E), lambda j: (0, 0))
    idx1, p1, keep1, g1, idx2, g2, keep2r, cntf, gsum = pl.pallas_call(
        lambda *refs: _routing_a(*refs, C=C),
        grid=(nb,),
        in_specs=[
            pl.BlockSpec((TB, D), lambda j: (j, 0)),
            pl.BlockSpec((D, E), lambda j: (0, 0)),
            tb_spec,
        ],
        out_specs=(tb_spec,) * 7 + (acc_spec, acc_spec),
        out_shape=outs_a,
        compiler_params=pltpu.CompilerParams(
            dimension_semantics=("arbitrary",)),
    )(x, w_gate, u2)

    p2, keep2, cnt2c = pl.pallas_call(
        lambda *refs: _routing_b(*refs, C=C),
        grid=(nb,),
        in_specs=[tb_spec, tb_spec, acc_spec],
        out_specs=(tb_spec, tb_spec, acc_spec),
        out_shape=(jax.ShapeDtypeStruct((T, 1), i32),
                   jax.ShapeDtypeStruct((T, 1), f32),
                   jax.ShapeDtypeStruct((1, E), f32)),
        scratch_shapes=[pltpu.VMEM((1, E), f32)],
        compiler_params=pltpu.CompilerParams(
            dimension_semantics=("arbitrary",)),
    )(idx2, keep2r, cntf)

    tid = jnp.arange(T, dtype=i32)
    i1 = idx1[:, 0]
    i2 = idx2[:, 0]
    p1f = p1[:, 0]
    p2f = p2[:, 0]
    e1s = jnp.where(keep1[:, 0] > 0, i1, E)
    e2s = jnp.where(keep2[:, 0] > 0, i2, E)
    assign = jnp.full((E, C), -1, i32)
    assign = assign.at[e1s, p1f].set(tid, mode="drop")
    assign = assign.at[e2s, p2f].set(tid, mode="drop")
    wslot = jnp.zeros((E, C), f32)
    wslot = wslot.at[e1s, p1f].set(g1[:, 0], mode="drop")
    wslot = wslot.at[e2s, p2f].set(g2[:, 0], mode="drop")
    af = assign.reshape(E * C)
    spread = (jnp.arange(E * C, dtype=i32) * 13) % T
    assign_flat = jnp.where(af >= 0, af, spread)
    xd = _make_sc_gather(E * C, T)(x, assign_flat)

    zrow = E * C
    odw = pl.pallas_call(
        _expert_ffn,
        grid=(E + 1,),
        in_specs=[
            pl.BlockSpec((C, D), lambda e: (e - e // E, 0)),
            pl.BlockSpec((1, C, 1), lambda e: (e - e // E, 0, 0)),
            pl.BlockSpec((1, D, DFF), lambda e: (e - e // E, 0, 0)),
            pl.BlockSpec((1, 1, DFF), lambda e: (e - e // E, 0, 0)),
            pl.BlockSpec((1, DFF, D), lambda e: (e - e // E, 0, 0)),
            pl.BlockSpec((1, 1, D), lambda e: (e - e // E, 0, 0)),
        ],
        out_specs=pl.BlockSpec((C, D), lambda e: (e, 0)),
        out_shape=jax.ShapeDtypeStruct(((E + 1) * C, D), f32),
        compiler_params=pltpu.CompilerParams(
            dimension_semantics=("arbitrary",)),
    )(xd, wslot.reshape(E, C, 1),
      W1, b1.reshape(E, 1, DFF),
      W2, b2.reshape(E, 1, D))

    loc1 = jnp.where(keep1[:, 0] > 0, i1 * C + p1f, zrow)
    loc2 = jnp.where(keep2[:, 0] > 0, i2 * C + p2f, zrow)
    locs = jnp.concatenate([loc1, loc2])
    y12 = _make_sc_gather(2 * T, (E + 1) * C)(odw, locs)
    yb = T // TB
    y = pl.pallas_call(
        _add_halves,
        grid=(yb,),
        in_specs=[
            pl.BlockSpec((TB, D), lambda j: (j, 0)),
            pl.BlockSpec((TB, D), lambda j: (j + yb, 0)),
        ],
        out_specs=pl.BlockSpec((TB, D), lambda j: (j, 0)),
        out_shape=jax.ShapeDtypeStruct((T, D), f32),
    )(y12, y12)

    density = cntf[0] / T
    density_proxy = gsum[0] / T
    loss = jnp.mean(density * density_proxy) * (E * E) * LOSS_COEF
    return y.reshape(orig_shape), loss
